# baseline (speedup 1.0000x reference)
import functools

import jax
import jax.numpy as jnp
from jax import lax
from jax.experimental import pallas as pl
from jax.experimental.pallas import tpu as pltpu
from jax.experimental.pallas import tpu_sc as plsc

_E, _N, _D, _R = 320000, 10000, 128, 16
_B = 80
_NROWS = _E // _B
_NW = 32
_NPT = _N // 16
_ZR = 25
_NBUF = 4
_NC = 2
_CHUNK_ROWS = (2000, 2000)


def _swish(v):
    return v * jax.nn.sigmoid(v)


_EB = 16000


def _edge_body(rbft_ref, x_ref, w_ref, o_ref):
    prod = (
        lax.dot_general(rbft_ref[...], w_ref[...],
                        dimension_numbers=(((0,), (0,)), ((), ())),
                        preferred_element_type=jnp.float32)
        * x_ref[...]
    )
    lo = lax.bitcast_convert_type(
        prod[:, :64].astype(jnp.bfloat16), jnp.uint16).astype(jnp.uint32)
    hi = lax.bitcast_convert_type(
        prod[:, 64:].astype(jnp.bfloat16), jnp.uint16).astype(jnp.uint32)
    packed = lax.bitcast_convert_type(lo | (hi << 16), jnp.int32)
    o_ref[...] = jnp.concatenate(
        [packed[: _EB // 2, :], packed[_EB // 2:, :]], axis=1)


def _edge_embed(rbf_t, x, W_rbf, e0, ce):
    nblk = ce // _EB
    off = e0 // _EB
    return pl.pallas_call(
        _edge_body,
        grid=(nblk,),
        in_specs=[
            pl.BlockSpec((_R, _EB), lambda i: (0, off + i)),
            pl.BlockSpec((_EB, _D), lambda i: (off + i, 0)),
            pl.BlockSpec((_R, _D), lambda i: (0, 0)),
        ],
        out_specs=pl.BlockSpec((_EB // 2, _D), lambda i: (i, 0)),
        out_shape=jax.ShapeDtypeStruct((ce // 2, _D), jnp.int32),
    )(rbf_t, x, W_rbf)


@functools.cache
def _make_scatter_kernel(row0, crows):
    mesh = plsc.VectorSubcoreMesh(core_axis_name="c", subcore_axis_name="s")
    wbase, wrem = crows // _NW, crows % _NW
    maxb = wbase + (1 if wrem else 0)
    outer = (maxb + _NBUF - 1) // _NBUF

    def _scatter_body(h_hbm, idx_hbm, out_hbm, idx_v, h_v, z_v, c_v, idx_s,
                      acc, sem, ssem):
        c = lax.axis_index("c")
        s = lax.axis_index("s")
        w = c * 16 + s

        def _zb(i, carry):
            z_v[i // 8, pl.ds((i % 8) * 16, 16)] = jnp.zeros((16,), jnp.float32)
            return carry

        lax.fori_loop(0, _ZR * 8, _zb, 0)
        base = s * _NPT
        for j in range(_NPT // _ZR):
            pltpu.async_copy(z_v, acc.at[pl.ds(base + j * _ZR, _ZR)],
                             sem.at[0])
        for j in range(_NPT // _ZR):
            pltpu.make_async_copy(z_v, acc.at[pl.ds(base + j * _ZR, _ZR)],
                                  sem.at[0]).wait()
        plsc.subcore_barrier()

        start = w * wbase + jnp.minimum(w, wrem)
        cnt = wbase + jnp.where(w < wrem, 1, 0)

        bpb = _EB // _B

        def _idx_offs(i):
            grow = row0 + start + i
            k = grow // bpb
            p = grow % bpb
            a0 = k * _EB + p * (_B // 2)
            return a0, a0 + _EB // 2

        def _fire(i, b):
            row = start + i
            a0, b0 = _idx_offs(i)
            pltpu.async_copy(idx_hbm.at[pl.ds(a0, _B // 2)],
                             idx_v.at[b, pl.ds(0, _B // 2)], sem.at[b])
            pltpu.async_copy(idx_hbm.at[pl.ds(b0, _B // 2)],
                             idx_v.at[b, pl.ds(_B // 2, _B // 2)], sem.at[b])
            pltpu.async_copy(h_hbm.at[row], h_v.at[b], sem.at[b])

        def _drain(i, b):
            row = start + i
            a0, b0 = _idx_offs(i)
            pltpu.make_async_copy(
                idx_hbm.at[pl.ds(a0, _B // 2)],
                idx_v.at[b, pl.ds(0, _B // 2)], sem.at[b]).wait()
            pltpu.make_async_copy(
                idx_hbm.at[pl.ds(b0, _B // 2)],
                idx_v.at[b, pl.ds(_B // 2, _B // 2)], sem.at[b]).wait()
            pltpu.make_async_copy(h_hbm.at[row], h_v.at[b], sem.at[b]).wait()

        for b in range(_NBUF):
            @pl.when(b < cnt)
            def _():
                _fire(b, b)

        def _body(j, carry):
            for b in range(_NBUF):
                i = j * _NBUF + b

                par = b % 2

                @pl.when(i < cnt)
                def _():
                    _drain(i, b)

                    @pl.when(i >= 2)
                    def _():
                        pltpu.make_async_copy(
                            c_v.at[par], acc.at[idx_s.at[par]],
                            ssem.at[par]).wait()

                    def _conv(r2, carry):
                        for g in range(8):
                            v = h_v[b, r2, pl.ds(g * 16, 16)]
                            e = r2 + (40 if g >= 4 else 0)
                            cb = (g % 4) * 16
                            c_v[par, e, pl.ds(cb, 16)] = plsc.bitcast(
                                v << 16, jnp.float32)
                            c_v[par, e, pl.ds(cb + 64, 16)] = plsc.bitcast(
                                v & jnp.int32(-65536), jnp.float32)
                        return carry

                    lax.fori_loop(0, _B // 2, _conv, 0)
                    for q in range(_B // 16):
                        idx_s[par, pl.ds(q * 16, 16)] = (
                            idx_v[b, pl.ds(q * 16, 16)])
                    pltpu.async_copy(c_v.at[par], acc.at[idx_s.at[par]],
                                     ssem.at[par], add=True)

                    @pl.when(i + _NBUF < cnt)
                    def _():
                        _fire(i + _NBUF, b)
            return carry

        lax.fori_loop(0, outer, _body, 0)
        for par in range(2):
            pltpu.make_async_copy(c_v.at[par], acc.at[idx_s.at[par]],
                                  ssem.at[par]).wait()
        plsc.subcore_barrier()

        pltpu.sync_copy(acc.at[pl.ds(base, _NPT)],
                        out_hbm.at[c, pl.ds(base, _NPT)])

    return functools.partial(
        pl.kernel,
        out_type=jax.ShapeDtypeStruct((2, _N, _D), jnp.float32),
        mesh=mesh,
        compiler_params=pltpu.CompilerParams(use_tc_tiling_on_sc=False,
                                             needs_layout_passes=False),
        scratch_types=[
            pltpu.VMEM((_NBUF, _B), jnp.int32),
            pltpu.VMEM((_NBUF, _B // 2, _D), jnp.int32),
            pltpu.VMEM((_ZR, _D), jnp.float32),
            pltpu.VMEM((2, _B, _D), jnp.float32),
            pltpu.VMEM((2, _B), jnp.int32),
            pltpu.VMEM_SHARED((_N, _D), jnp.float32),
            pltpu.SemaphoreType.DMA((_NBUF,)),
            pltpu.SemaphoreType.DMA((2,)),
        ],
    )(_scatter_body)


_NB = 1000


def _mlp_body(*refs):
    p_refs = refs[:_NC]
    w1_ref, b1_ref, w2_ref, b2_ref, w3_ref, b3_ref, wo_ref, o_ref = refs[_NC:]
    agg = p_refs[0][0] + p_refs[0][1]
    for p in p_refs[1:]:
        agg = agg + p[0] + p[1]
    h = _swish(jnp.dot(agg, w1_ref[...], preferred_element_type=jnp.float32)
               + b1_ref[...])
    h = _swish(jnp.dot(h, w2_ref[...], preferred_element_type=jnp.float32)
               + b2_ref[...])
    h = _swish(jnp.dot(h, w3_ref[...], preferred_element_type=jnp.float32)
               + b3_ref[...])
    o_ref[...] = jnp.dot(h, wo_ref[...], preferred_element_type=jnp.float32)


def _mlp(partial_list, W1, b1, W2, b2, W3, b3, W_out):
    O = W_out.shape[1]
    return pl.pallas_call(
        _mlp_body,
        grid=(_N // _NB,),
        in_specs=(
            [pl.BlockSpec((2, _NB, _D), lambda i: (0, i, 0))] * _NC
            + [
                pl.BlockSpec((_D, _D), lambda i: (0, 0)),
                pl.BlockSpec((1, _D), lambda i: (0, 0)),
                pl.BlockSpec((_D, _D), lambda i: (0, 0)),
                pl.BlockSpec((1, _D), lambda i: (0, 0)),
                pl.BlockSpec((_D, _D), lambda i: (0, 0)),
                pl.BlockSpec((1, _D), lambda i: (0, 0)),
                pl.BlockSpec((_D, O), lambda i: (0, 0)),
            ]
        ),
        out_specs=pl.BlockSpec((_NB, O), lambda i: (i, 0)),
        out_shape=jax.ShapeDtypeStruct((_N, O), jnp.float32),
    )(*partial_list, W1, b1.reshape(1, _D), W2, b2.reshape(1, _D), W3,
      b3.reshape(1, _D), W_out)


def kernel(x, rbf, idx_i, num_nodes, W_rbf, W1, b1, W2, b2, W3, b3, W_out):
    idx = idx_i.astype(jnp.int32)
    rbf_t = rbf.T
    partial_list = []
    row0 = 0
    for crows in _CHUNK_ROWS:
        h_k = _edge_embed(rbf_t, x, W_rbf, row0 * _B, crows * _B)
        partial_list.append(
            _make_scatter_kernel(row0, crows)(
                h_k.reshape(crows, _B // 2, _D), idx))
        row0 += crows
    return _mlp(partial_list, W1, b1, W2, b2, W3, b3, W_out)


# speedup vs baseline: 1.0044x; 1.0044x over previous
import functools

import jax
import jax.numpy as jnp
from jax import lax
from jax.experimental import pallas as pl
from jax.experimental.pallas import tpu as pltpu
from jax.experimental.pallas import tpu_sc as plsc

_E, _N, _D, _R = 320000, 10000, 128, 16
_B = 80
_NROWS = _E // _B
_NW = 32
_NPT = _N // 16
_ZR = 25
_NBUF = 4
_NC = 2
_CHUNK_ROWS = (2000, 2000)


def _swish(v):
    return v * jax.nn.sigmoid(v)


_EB = 16000


def _edge_body(rbft_ref, x_ref, w_ref, o_ref):
    prod = (
        lax.dot_general(rbft_ref[...], w_ref[...],
                        dimension_numbers=(((0,), (0,)), ((), ())),
                        preferred_element_type=jnp.float32)
        * x_ref[...]
    )
    lo = lax.bitcast_convert_type(
        prod[:, :64].astype(jnp.bfloat16), jnp.uint16).astype(jnp.uint32)
    hi = lax.bitcast_convert_type(
        prod[:, 64:].astype(jnp.bfloat16), jnp.uint16).astype(jnp.uint32)
    packed = lax.bitcast_convert_type(lo | (hi << 16), jnp.int32)
    o_ref[...] = jnp.concatenate(
        [packed[: _EB // 2, :], packed[_EB // 2:, :]], axis=1)


def _edge_embed(rbf_t, x, W_rbf, e0, ce):
    nblk = ce // _EB
    off = e0 // _EB
    return pl.pallas_call(
        _edge_body,
        grid=(nblk,),
        in_specs=[
            pl.BlockSpec((_R, _EB), lambda i: (0, off + i)),
            pl.BlockSpec((_EB, _D), lambda i: (off + i, 0)),
            pl.BlockSpec((_R, _D), lambda i: (0, 0)),
        ],
        out_specs=pl.BlockSpec((_EB // 2, _D), lambda i: (i, 0)),
        out_shape=jax.ShapeDtypeStruct((ce // 2, _D), jnp.int32),
    )(rbf_t, x, W_rbf)


@functools.cache
def _make_scatter_kernel(row0, crows):
    mesh = plsc.VectorSubcoreMesh(core_axis_name="c", subcore_axis_name="s")
    wbase, wrem = crows // _NW, crows % _NW
    maxb = wbase + (1 if wrem else 0)
    outer = (maxb + _NBUF - 1) // _NBUF

    def _scatter_body(h_hbm, idx_hbm, out_hbm, idx_v, h_v, z_v, c_v, idx_s,
                      acc, sem, ssem):
        c = lax.axis_index("c")
        s = lax.axis_index("s")
        w = c * 16 + s

        def _zb(i, carry):
            z_v[i // 8, pl.ds((i % 8) * 16, 16)] = jnp.zeros((16,), jnp.float32)
            return carry

        lax.fori_loop(0, _ZR * 8, _zb, 0)
        base = s * _NPT
        for j in range(_NPT // _ZR):
            pltpu.async_copy(z_v, acc.at[pl.ds(base + j * _ZR, _ZR)],
                             sem.at[0])
        for j in range(_NPT // _ZR):
            pltpu.make_async_copy(z_v, acc.at[pl.ds(base + j * _ZR, _ZR)],
                                  sem.at[0]).wait()
        plsc.subcore_barrier()

        start = w * wbase + jnp.minimum(w, wrem)
        cnt = wbase + jnp.where(w < wrem, 1, 0)

        bpb = _EB // _B

        def _idx_offs(i):
            grow = row0 + start + i
            k = grow // bpb
            p = grow % bpb
            a0 = k * _EB + p * (_B // 2)
            return a0, a0 + _EB // 2

        def _fire(i, b):
            row = start + i
            a0, b0 = _idx_offs(i)
            pltpu.async_copy(idx_hbm.at[pl.ds(a0, _B // 2)],
                             idx_v.at[b, pl.ds(0, _B // 2)], sem.at[b])
            pltpu.async_copy(idx_hbm.at[pl.ds(b0, _B // 2)],
                             idx_v.at[b, pl.ds(_B // 2, _B // 2)], sem.at[b])
            pltpu.async_copy(h_hbm.at[row], h_v.at[b], sem.at[b])

        def _drain(i, b):
            row = start + i
            a0, b0 = _idx_offs(i)
            pltpu.make_async_copy(
                idx_hbm.at[pl.ds(a0, _B // 2)],
                idx_v.at[b, pl.ds(0, _B // 2)], sem.at[b]).wait()
            pltpu.make_async_copy(
                idx_hbm.at[pl.ds(b0, _B // 2)],
                idx_v.at[b, pl.ds(_B // 2, _B // 2)], sem.at[b]).wait()
            pltpu.make_async_copy(h_hbm.at[row], h_v.at[b], sem.at[b]).wait()

        for b in range(_NBUF):
            @pl.when(b < cnt)
            def _():
                _fire(b, b)

        def _body(j, carry):
            for b in range(_NBUF):
                i = j * _NBUF + b

                par = b % 2

                @pl.when(i < cnt)
                def _():
                    _drain(i, b)

                    @pl.when(i >= 2)
                    def _():
                        pltpu.make_async_copy(
                            c_v.at[par], acc.at[idx_s.at[par]],
                            ssem.at[par]).wait()

                    def _conv(r2, carry):
                        for g in range(8):
                            v = h_v[b, r2, pl.ds(g * 16, 16)]
                            e = r2 + (40 if g >= 4 else 0)
                            cb = (g % 4) * 16
                            c_v[par, e, pl.ds(cb, 16)] = plsc.bitcast(
                                v << 16, jnp.float32)
                            c_v[par, e, pl.ds(cb + 64, 16)] = plsc.bitcast(
                                v & jnp.int32(-65536), jnp.float32)
                        return carry

                    lax.fori_loop(0, _B // 2, _conv, 0)
                    for q in range(_B // 16):
                        idx_s[par, pl.ds(q * 16, 16)] = (
                            idx_v[b, pl.ds(q * 16, 16)])
                    pltpu.async_copy(c_v.at[par], acc.at[idx_s.at[par]],
                                     ssem.at[par], add=True)

                    @pl.when(i + _NBUF < cnt)
                    def _():
                        _fire(i + _NBUF, b)
            return carry

        lax.fori_loop(0, outer, _body, 0)
        for par in range(2):
            pltpu.make_async_copy(c_v.at[par], acc.at[idx_s.at[par]],
                                  ssem.at[par]).wait()
        plsc.subcore_barrier()

        pltpu.sync_copy(acc.at[pl.ds(base, _NPT)],
                        out_hbm.at[c, pl.ds(base, _NPT)])

    return functools.partial(
        pl.kernel,
        out_type=jax.ShapeDtypeStruct((2, _N, _D), jnp.float32),
        mesh=mesh,
        compiler_params=pltpu.CompilerParams(use_tc_tiling_on_sc=False,
                                             needs_layout_passes=False),
        scratch_types=[
            pltpu.VMEM((_NBUF, _B), jnp.int32),
            pltpu.VMEM((_NBUF, _B // 2, _D), jnp.int32),
            pltpu.VMEM((_ZR, _D), jnp.float32),
            pltpu.VMEM((2, _B, _D), jnp.float32),
            pltpu.VMEM((2, _B), jnp.int32),
            pltpu.VMEM_SHARED((_N, _D), jnp.float32),
            pltpu.SemaphoreType.DMA((_NBUF,)),
            pltpu.SemaphoreType.DMA((2,)),
        ],
    )(_scatter_body)


_NB = 1000


def _padd_body(p_ref, o_ref):
    o_ref[...] = p_ref[0] + p_ref[1]


def _padd(partials):
    return pl.pallas_call(
        _padd_body,
        grid=(_N // _NB,),
        in_specs=[pl.BlockSpec((2, _NB, _D), lambda i: (0, i, 0))],
        out_specs=pl.BlockSpec((_NB, _D), lambda i: (i, 0)),
        out_shape=jax.ShapeDtypeStruct((_N, _D), jnp.float32),
    )(partials)


def _mlp_body(*refs):
    pa_ref, pb_ref = refs[:2]
    w1_ref, b1_ref, w2_ref, b2_ref, w3_ref, b3_ref, wo_ref, o_ref = refs[2:]
    agg = pa_ref[...] + pb_ref[0] + pb_ref[1]
    h = _swish(jnp.dot(agg, w1_ref[...], preferred_element_type=jnp.float32)
               + b1_ref[...])
    h = _swish(jnp.dot(h, w2_ref[...], preferred_element_type=jnp.float32)
               + b2_ref[...])
    h = _swish(jnp.dot(h, w3_ref[...], preferred_element_type=jnp.float32)
               + b3_ref[...])
    o_ref[...] = jnp.dot(h, wo_ref[...], preferred_element_type=jnp.float32)


def _mlp(partial_list, W1, b1, W2, b2, W3, b3, W_out):
    O = W_out.shape[1]
    return pl.pallas_call(
        _mlp_body,
        grid=(_N // _NB,),
        in_specs=(
            [
                pl.BlockSpec((_NB, _D), lambda i: (i, 0)),
                pl.BlockSpec((2, _NB, _D), lambda i: (0, i, 0)),
            ]
            + [
                pl.BlockSpec((_D, _D), lambda i: (0, 0)),
                pl.BlockSpec((1, _D), lambda i: (0, 0)),
                pl.BlockSpec((_D, _D), lambda i: (0, 0)),
                pl.BlockSpec((1, _D), lambda i: (0, 0)),
                pl.BlockSpec((_D, _D), lambda i: (0, 0)),
                pl.BlockSpec((1, _D), lambda i: (0, 0)),
                pl.BlockSpec((_D, O), lambda i: (0, 0)),
            ]
        ),
        out_specs=pl.BlockSpec((_NB, O), lambda i: (i, 0)),
        out_shape=jax.ShapeDtypeStruct((_N, O), jnp.float32),
    )(*partial_list, W1, b1.reshape(1, _D), W2, b2.reshape(1, _D), W3,
      b3.reshape(1, _D), W_out)


def kernel(x, rbf, idx_i, num_nodes, W_rbf, W1, b1, W2, b2, W3, b3, W_out):
    idx = idx_i.astype(jnp.int32)
    rbf_t = rbf.T
    partial_list = []
    row0 = 0
    for crows in _CHUNK_ROWS:
        h_k = _edge_embed(rbf_t, x, W_rbf, row0 * _B, crows * _B)
        partial_list.append(
            _make_scatter_kernel(row0, crows)(
                h_k.reshape(crows, _B // 2, _D), idx))
        row0 += crows
    partial_list[0] = _padd(partial_list[0])
    return _mlp(partial_list, W1, b1, W2, b2, W3, b3, W_out)
